# baseline (speedup 1.0000x reference)
import jax
import jax.numpy as jnp
from jax.experimental import pallas as pl

TOKEN = 32
HID = 768
SBLK = 256


def _emb_kernel(bits_ref, table_ref, emb_ref, logit_ref):
    amp = bits_ref[...] * 2.0 - 1.0
    tab = table_ref[...]
    emb_ref[...] = amp[:, :, None] * tab[None, :, :]
    rowsum = jnp.sum(tab, axis=1)
    logit_ref[...] = amp * rowsum[None, :]


def kernel(text_batch, table):
    B, flat = text_batch.shape
    S = flat // TOKEN
    N = B * S
    bits = text_batch.reshape(N, TOKEN)
    emb, logit = pl.pallas_call(
        _emb_kernel,
        grid=(N // SBLK,),
        in_specs=[
            pl.BlockSpec((SBLK, TOKEN), lambda i: (i, 0)),
            pl.BlockSpec((TOKEN, HID), lambda i: (0, 0)),
        ],
        out_specs=[
            pl.BlockSpec((SBLK, TOKEN, HID), lambda i: (i, 0, 0)),
            pl.BlockSpec((SBLK, TOKEN), lambda i: (i, 0)),
        ],
        out_shape=[
            jax.ShapeDtypeStruct((N, TOKEN, HID), jnp.float32),
            jax.ShapeDtypeStruct((N, TOKEN), jnp.float32),
        ],
    )(bits, table)
    return emb.reshape(B, S, TOKEN, HID), logit.reshape(B, S, TOKEN, 1)


# speedup vs baseline: 1.0366x; 1.0366x over previous
import jax
import jax.numpy as jnp
from jax.experimental import pallas as pl

TOKEN = 32
HID = 768
SBLK = 64


def _emb_kernel(bits_ref, table_ref, emb_ref, logit_ref):
    amp = bits_ref[...] * 2.0 - 1.0
    tab = table_ref[...]
    emb_ref[...] = amp[:, :, None] * tab[None, :, :]
    rowsum = jnp.sum(tab, axis=1)
    logit_ref[...] = amp * rowsum[None, :]


def kernel(text_batch, table):
    B, flat = text_batch.shape
    S = flat // TOKEN
    N = B * S
    bits = text_batch.reshape(N, TOKEN)
    emb, logit = pl.pallas_call(
        _emb_kernel,
        grid=(N // SBLK,),
        in_specs=[
            pl.BlockSpec((SBLK, TOKEN), lambda i: (i, 0)),
            pl.BlockSpec((TOKEN, HID), lambda i: (0, 0)),
        ],
        out_specs=[
            pl.BlockSpec((SBLK, TOKEN, HID), lambda i: (i, 0, 0)),
            pl.BlockSpec((SBLK, TOKEN), lambda i: (i, 0)),
        ],
        out_shape=[
            jax.ShapeDtypeStruct((N, TOKEN, HID), jnp.float32),
            jax.ShapeDtypeStruct((N, TOKEN), jnp.float32),
        ],
    )(bits, table)
    return emb.reshape(B, S, TOKEN, HID), logit.reshape(B, S, TOKEN, 1)
